# baseline (speedup 1.0000x reference)
import functools

import jax
import jax.numpy as jnp
from jax import lax
from jax.experimental import pallas as pl
from jax.experimental.pallas import tpu as pltpu
from jax.experimental.pallas import tpu_sc as plsc


def kernel(x, pos_embedding):
    B, S, D = x.shape
    info = plsc.get_sparse_core_info()
    NC, NS = info.num_cores, info.num_subcores
    NW = NC * NS
    SPW = S // NW
    C = 32
    NCH = SPW // C
    VPR = D // 16

    x2 = x.reshape(B * S, D)
    mesh = plsc.VectorSubcoreMesh(core_axis_name="c", subcore_axis_name="s")

    @functools.partial(
        pl.kernel,
        mesh=mesh,
        out_type=jax.ShapeDtypeStruct((B * S, D), jnp.float32),
        scratch_types=[
            pltpu.VMEM((C, D), jnp.float32),
            pltpu.VMEM((C, D), jnp.float32),
        ],
    )
    def sc_add(x_hbm, pos_hbm, out_hbm, pbuf, xbuf):
        wid = lax.axis_index("s") * NC + lax.axis_index("c")
        seq0 = wid * SPW

        def chunk_body(c, _):
            pltpu.sync_copy(pos_hbm.at[pl.ds(seq0 + c * C, C)], pbuf)

            def batch_body(b, _):
                row0 = b * S + seq0 + c * C
                pltpu.sync_copy(x_hbm.at[pl.ds(row0, C)], xbuf)

                def vec_body(k, _):
                    r = k // VPR
                    j = (k % VPR) * 16
                    xbuf[r, pl.ds(j, 16)] = xbuf[r, pl.ds(j, 16)] + pbuf[r, pl.ds(j, 16)]
                    return 0

                lax.fori_loop(0, C * VPR, vec_body, 0)
                pltpu.sync_copy(xbuf, out_hbm.at[pl.ds(row0, C)])
                return 0

            lax.fori_loop(0, B, batch_body, 0)
            return 0

        lax.fori_loop(0, NCH, chunk_body, 0)

    out = sc_add(x2, pos_embedding)
    return out.reshape(B, S, D)


# speedup vs baseline: 2.5159x; 2.5159x over previous
import functools

import jax
import jax.numpy as jnp
from jax import lax
from jax.experimental import pallas as pl
from jax.experimental.pallas import tpu as pltpu
from jax.experimental.pallas import tpu_sc as plsc


def kernel(x, pos_embedding):
    B, S, D = x.shape
    info = plsc.get_sparse_core_info()
    NC, NS = info.num_cores, info.num_subcores
    NW = NC * NS
    SPW = S // NW
    C = 16
    NCH = SPW // C
    T = NCH * B
    VPR = D // 16

    x2 = x.reshape(B * S, D)
    mesh = plsc.VectorSubcoreMesh(core_axis_name="c", subcore_axis_name="s")

    @functools.partial(
        pl.kernel,
        mesh=mesh,
        out_type=jax.ShapeDtypeStruct((B * S, D), jnp.float32),
        scratch_types=[
            pltpu.VMEM((C, D), jnp.float32),
            pltpu.VMEM((C, D), jnp.float32),
            pltpu.VMEM((C, D), jnp.float32),
            pltpu.VMEM((C, D), jnp.float32),
            pltpu.SemaphoreType.DMA,
            pltpu.SemaphoreType.DMA,
            pltpu.SemaphoreType.DMA,
        ],
    )
    def sc_add(x_hbm, pos_hbm, out_hbm, xb0, xb1, pb0, pb1, sin, sout, spos):
        wid = lax.axis_index("s") * NC + lax.axis_index("c")
        seq0 = wid * SPW
        xbufs = (xb0, xb1)
        pbufs = (pb0, pb1)

        def row0(t):
            c, b = divmod(t, B)
            return b * S + seq0 + c * C

        h_in = [None] * T
        h_out = [None] * T
        h_pos = [None] * NCH

        h_pos[0] = pltpu.async_copy(
            pos_hbm.at[pl.ds(seq0, C)], pbufs[0], spos
        )
        h_in[0] = pltpu.async_copy(
            x_hbm.at[pl.ds(row0(0), C)], xbufs[0], sin
        )

        for t in range(T):
            c, b = divmod(t, B)
            xb = xbufs[t % 2]
            pb = pbufs[c % 2]
            if t + 1 < T:
                if t >= 1:
                    h_out[t - 1].wait()
                h_in[t + 1] = pltpu.async_copy(
                    x_hbm.at[pl.ds(row0(t + 1), C)],
                    xbufs[(t + 1) % 2],
                    sin,
                )
            if b == 0:
                if c + 1 < NCH:
                    h_pos[c + 1] = pltpu.async_copy(
                        pos_hbm.at[pl.ds(seq0 + (c + 1) * C, C)],
                        pbufs[(c + 1) % 2],
                        spos,
                    )
                h_pos[c].wait()
            h_in[t].wait()

            def vec_body(g, _):
                r = g // 2
                h = (g % 2) * (VPR // 2) * 16
                for j in range(VPR // 2):
                    sl = pl.ds(h + j * 16, 16)
                    xb[r, sl] = xb[r, sl] + pb[r, sl]
                return 0

            lax.fori_loop(0, C * 2, vec_body, 0)
            h_out[t] = pltpu.async_copy(
                xb, out_hbm.at[pl.ds(row0(t), C)], sout
            )
        h_out[T - 2].wait()
        h_out[T - 1].wait()

    out = sc_add(x2, pos_embedding)
    return out.reshape(B, S, D)
